# baseline (speedup 1.0000x reference)
import functools

import jax
import jax.numpy as jnp
from jax import lax
from jax.experimental import pallas as pl
from jax.experimental.pallas import tpu as pltpu
from jax.experimental.pallas import tpu_sc as plsc

N = 10000
C = 128
K = 27
P = 12000

NPAD = 10240
NW = 32
TILE_ROWS = NPAD // 16
DUMMY_ROW = N
CH = 64

KA = 13
KB = K - KA
STEPS_A = 80
STEPS_B = 84

_MM_BM = 1000


def _mm1_body(nk, x_ref, w_ref, o_ref):
    for t in range(nk):
        o_ref[t] = jnp.dot(x_ref[...], w_ref[:, t * C:(t + 1) * C],
                           preferred_element_type=jnp.float32)


def _mm1(x, w, nk):
    return pl.pallas_call(
        functools.partial(_mm1_body, nk),
        grid=(N // _MM_BM,),
        in_specs=[
            pl.BlockSpec((_MM_BM, C), lambda i: (i, 0)),
            pl.BlockSpec((C, nk * C), lambda i: (0, 0)),
        ],
        out_specs=pl.BlockSpec((nk, _MM_BM, C), lambda i: (0, i, 0)),
        out_shape=jax.ShapeDtypeStruct((nk, N, C), jnp.float32),
    )(x, w)


def _mm2_body(nk, pa_ref, pb_ref, b_ref, w_ref, o_ref):
    h = jnp.maximum(pa_ref[0] + pa_ref[1] + pb_ref[0] + pb_ref[1]
                    + b_ref[...], 0.0)
    for t in range(nk):
        o_ref[t] = jnp.dot(h, w_ref[:, t * C:(t + 1) * C],
                           preferred_element_type=jnp.float32)


def _mm2(pa, pb, b, w, nk):
    pblk = pl.BlockSpec((2, _MM_BM, C), lambda i: (0, i, 0))
    return pl.pallas_call(
        functools.partial(_mm2_body, nk),
        grid=(N // _MM_BM,),
        in_specs=[
            pblk, pblk,
            pl.BlockSpec((1, C), lambda i: (0, 0)),
            pl.BlockSpec((C, nk * C), lambda i: (0, 0)),
        ],
        out_specs=pl.BlockSpec((nk, _MM_BM, C), lambda i: (0, i, 0)),
        out_shape=jax.ShapeDtypeStruct((nk, N, C), jnp.float32),
    )(pa, pb, b, w)


def _final_body(pa_ref, pb_ref, b_ref, f_ref, o_ref):
    o_ref[...] = jnp.maximum(
        pa_ref[0] + pa_ref[1] + pb_ref[0] + pb_ref[1]
        + b_ref[...] + f_ref[...], 0.0)


def _final(pa, pb, b, f):
    blk = pl.BlockSpec((1000, C), lambda i: (i, 0))
    pblk = pl.BlockSpec((2, 1000, C), lambda i: (0, i, 0))
    return pl.pallas_call(
        _final_body,
        grid=(N // 1000,),
        in_specs=[pblk, pblk,
                  pl.BlockSpec((1, C), lambda i: (0, 0)), blk],
        out_specs=blk,
        out_shape=jax.ShapeDtypeStruct((N, C), jnp.float32),
    )(pa, pb, b, f)


def _make_sc(steps):
    blocks = steps // 2

    @functools.partial(
        pl.kernel,
        out_type=jax.ShapeDtypeStruct((2, NPAD, C), jnp.float32),
        mesh=plsc.VectorSubcoreMesh(core_axis_name="c",
                                    subcore_axis_name="s"),
        scratch_types=[
            pltpu.VMEM((3, 2, 2, CH), jnp.int32),
            pltpu.VMEM((4, CH, C), jnp.float32),
            pltpu.VMEM_SHARED((NPAD, C), jnp.float32),
            pltpu.SemaphoreType.DMA,
            pltpu.SemaphoreType.DMA,
            pltpu.SemaphoreType.DMA,
            pltpu.SemaphoreType.DMA,
            pltpu.SemaphoreType.DMA,
            pltpu.SemaphoreType.DMA,
            pltpu.SemaphoreType.DMA,
            pltpu.SemaphoreType.DMA,
            pltpu.SemaphoreType.DMA,
        ],
    )
    def _sc_gather_scatter(y_hbm, idx_hbm, zeros_hbm, out_hbm,
                           iio, rows, accum, isem,
                           g0, g1, g2, g3, s0, s1, s2, s3):
        gsems = (g0, g1, g2, g3)
        ssems = (s0, s1, s2, s3)
        cid = lax.axis_index("c")
        sid = lax.axis_index("s")
        wid = sid * 2 + cid

        pltpu.sync_copy(zeros_hbm,
                        accum.at[pl.ds(sid * TILE_ROWS, TILE_ROWS)])
        plsc.subcore_barrier()

        base = wid * steps

        def drain(sem_b):
            pltpu.make_async_copy(y_hbm.at[pl.ds(0, CH)], rows.at[0],
                                  sem_b).wait()

        def drain_idx():
            pltpu.make_async_copy(idx_hbm.at[pl.ds(0, 2)], iio.at[0],
                                  isem).wait()

        def issue_gathers(sl, p):
            for i in range(2):
                b = 2 * p + i
                pltpu.async_copy(y_hbm.at[iio.at[sl, i, 0]], rows.at[b],
                                 gsems[b])

        def issue_idx(next_blk, next_sl):
            pltpu.async_copy(idx_hbm.at[pl.ds(base + 2 * next_blk, 2)],
                             iio.at[next_sl], isem)

        def issue_scatters(sl, p):
            for i in range(2):
                b = 2 * p + i
                pltpu.async_copy(rows.at[b], accum.at[iio.at[sl, i, 1]],
                                 ssems[b], add=True)

        pltpu.sync_copy(idx_hbm.at[pl.ds(base, 2)], iio.at[0])
        issue_gathers(0, 0)
        issue_idx(1, 1)
        drain_idx()
        issue_gathers(1, 1)
        issue_idx(2, 2)
        drain(g0)
        drain(g1)
        issue_scatters(0, 0)

        def superblock(sb, carry):
            for p in range(2):
                blk = 2 * sb + p
                sl = lax.rem(blk, 3)
                slp = lax.rem(blk - 1, 3)
                q = 1 - p
                drain(ssems[2 * p])
                drain(ssems[2 * p + 1])
                drain_idx()
                issue_gathers(sl, p)
                issue_idx(blk + 1, lax.rem(blk + 1, 3))
                drain(gsems[2 * q])
                drain(gsems[2 * q + 1])
                issue_scatters(slp, q)
            return carry

        lax.fori_loop(1, blocks // 2, superblock, 0)

        drain(g2)
        drain(g3)
        issue_scatters((blocks - 1) % 3, 1)
        for b in range(4):
            drain(ssems[b])
        drain_idx()

        plsc.subcore_barrier()
        pltpu.sync_copy(accum.at[pl.ds(sid * TILE_ROWS, TILE_ROWS)],
                        out_hbm.at[cid, pl.ds(sid * TILE_ROWS, TILE_ROWS)])

    return _sc_gather_scatter


_SC_A = _make_sc(STEPS_A)
_SC_B = _make_sc(STEPS_B)


def _slice_idx(p_in, p_out, k0, nk, steps):
    cap = NW * steps * CH
    kp = nk * P
    npad_pairs = cap - kp
    pad_ramp = jnp.arange(npad_pairs, dtype=jnp.int32)
    iin = (p_in[k0:k0 + nk]
           + (jnp.arange(nk, dtype=jnp.int32) * N)[:, None]).reshape(-1)
    iin = jnp.concatenate([iin, (pad_ramp * 4099) % (nk * N)])
    iout = jnp.concatenate([p_out[k0:k0 + nk].reshape(-1),
                            DUMMY_ROW + pad_ramp % (NPAD - N)])
    idx = jnp.stack([iin.reshape(NW * steps, CH),
                     iout.reshape(NW * steps, CH)], axis=1)
    return jnp.pad(idx, ((0, 2), (0, 0), (0, 0)))


def kernel(feats, pairs_in, pairs_out, W1, g1, b1, m1, v1,
           W2, g2, b2, m2, v2):
    eps = 1e-5
    s1 = g1 * lax.rsqrt(v1 + eps)
    s2 = g2 * lax.rsqrt(v2 + eps)
    w1c = (W1 * s1[None, None, :]).transpose(1, 0, 2).reshape(C, K * C)
    w2c = (W2 * s2[None, None, :]).transpose(1, 0, 2).reshape(C, K * C)
    b1e = (b1 - m1 * s1).reshape(1, C)
    b2e = (b2 - m2 * s2).reshape(1, C)

    idx_a = _slice_idx(pairs_in, pairs_out, 0, KA, STEPS_A)
    idx_b = _slice_idx(pairs_in, pairs_out, KA, KB, STEPS_B)
    zeros = jnp.zeros((TILE_ROWS, C), dtype=jnp.float32)

    y1a = _mm1(feats, w1c[:, :KA * C], KA).reshape(KA * N, C)
    p1a = _SC_A(y1a, idx_a, zeros)
    y1b = _mm1(feats, w1c[:, KA * C:], KB).reshape(KB * N, C)
    p1b = _SC_B(y1b, idx_b, zeros)

    y2a = _mm2(p1a, p1b, b1e, w2c[:, :KA * C], KA).reshape(KA * N, C)
    p2a = _SC_A(y2a, idx_a, zeros)
    y2b = _mm2(p1a, p1b, b1e, w2c[:, KA * C:], KB).reshape(KB * N, C)
    p2b = _SC_B(y2b, idx_b, zeros)

    return _final(p2a, p2b, b2e, feats)


# speedup vs baseline: 1.0083x; 1.0083x over previous
import functools

import jax
import jax.numpy as jnp
from jax import lax
from jax.experimental import pallas as pl
from jax.experimental.pallas import tpu as pltpu
from jax.experimental.pallas import tpu_sc as plsc

N = 10000
C = 128
K = 27
P = 12000

NPAD = 10240
NW = 32
TILE_ROWS = NPAD // 16
DUMMY_ROW = N
CH = 64

KA = 13
KB = K - KA
STEPS_A = 80
STEPS_B = 84

_MM_BM = 1024


def _mm1_body(nk, x_ref, w_ref, o_ref):
    for t in range(nk):
        o_ref[t] = jnp.dot(x_ref[...], w_ref[:, t * C:(t + 1) * C],
                           preferred_element_type=jnp.float32)


def _mm1(x, w, nk):
    return pl.pallas_call(
        functools.partial(_mm1_body, nk),
        grid=(NPAD // _MM_BM,),
        in_specs=[
            pl.BlockSpec((_MM_BM, C), lambda i: (i, 0)),
            pl.BlockSpec((C, nk * C), lambda i: (0, 0)),
        ],
        out_specs=pl.BlockSpec((nk, _MM_BM, C), lambda i: (0, i, 0)),
        out_shape=jax.ShapeDtypeStruct((nk, NPAD, C), jnp.float32),
    )(x, w)


def _mm2_body(nk, pa_ref, pb_ref, b_ref, w_ref, o_ref):
    h = jnp.maximum(pa_ref[0] + pa_ref[1] + pb_ref[0] + pb_ref[1]
                    + b_ref[...], 0.0)
    for t in range(nk):
        o_ref[t] = jnp.dot(h, w_ref[:, t * C:(t + 1) * C],
                           preferred_element_type=jnp.float32)


def _mm2(pa, pb, b, w, nk):
    pblk = pl.BlockSpec((2, _MM_BM, C), lambda i: (0, i, 0))
    return pl.pallas_call(
        functools.partial(_mm2_body, nk),
        grid=(NPAD // _MM_BM,),
        in_specs=[
            pblk, pblk,
            pl.BlockSpec((1, C), lambda i: (0, 0)),
            pl.BlockSpec((C, nk * C), lambda i: (0, 0)),
        ],
        out_specs=pl.BlockSpec((nk, _MM_BM, C), lambda i: (0, i, 0)),
        out_shape=jax.ShapeDtypeStruct((nk, NPAD, C), jnp.float32),
    )(pa, pb, b, w)


def _final_body(pa_ref, pb_ref, b_ref, f_ref, o_ref):
    o_ref[...] = jnp.maximum(
        pa_ref[0] + pa_ref[1] + pb_ref[0] + pb_ref[1]
        + b_ref[...] + f_ref[...], 0.0)


def _final(pa, pb, b, f):
    blk = pl.BlockSpec((1000, C), lambda i: (i, 0))
    pblk = pl.BlockSpec((2, 1000, C), lambda i: (0, i, 0))
    return pl.pallas_call(
        _final_body,
        grid=(N // 1000,),
        in_specs=[pblk, pblk,
                  pl.BlockSpec((1, C), lambda i: (0, 0)), blk],
        out_specs=blk,
        out_shape=jax.ShapeDtypeStruct((N, C), jnp.float32),
    )(pa, pb, b, f)


def _make_sc(steps):
    blocks = steps // 2

    @functools.partial(
        pl.kernel,
        out_type=jax.ShapeDtypeStruct((2, NPAD, C), jnp.float32),
        mesh=plsc.VectorSubcoreMesh(core_axis_name="c",
                                    subcore_axis_name="s"),
        scratch_types=[
            pltpu.VMEM((3, 2, 2, CH), jnp.int32),
            pltpu.VMEM((4, CH, C), jnp.float32),
            pltpu.VMEM_SHARED((NPAD, C), jnp.float32),
            pltpu.SemaphoreType.DMA,
            pltpu.SemaphoreType.DMA,
            pltpu.SemaphoreType.DMA,
            pltpu.SemaphoreType.DMA,
            pltpu.SemaphoreType.DMA,
            pltpu.SemaphoreType.DMA,
            pltpu.SemaphoreType.DMA,
            pltpu.SemaphoreType.DMA,
            pltpu.SemaphoreType.DMA,
        ],
    )
    def _sc_gather_scatter(y_hbm, idx_hbm, zeros_hbm, out_hbm,
                           iio, rows, accum, isem,
                           g0, g1, g2, g3, s0, s1, s2, s3):
        gsems = (g0, g1, g2, g3)
        ssems = (s0, s1, s2, s3)
        cid = lax.axis_index("c")
        sid = lax.axis_index("s")
        wid = sid * 2 + cid

        pltpu.sync_copy(zeros_hbm.at[pl.ds(sid * TILE_ROWS, TILE_ROWS)],
                        accum.at[pl.ds(sid * TILE_ROWS, TILE_ROWS)])
        plsc.subcore_barrier()

        base = wid * steps

        def drain(sem_b):
            pltpu.make_async_copy(y_hbm.at[pl.ds(0, CH)], rows.at[0],
                                  sem_b).wait()

        def drain_idx():
            pltpu.make_async_copy(idx_hbm.at[pl.ds(0, 2)], iio.at[0],
                                  isem).wait()

        def issue_gathers(sl, p):
            for i in range(2):
                b = 2 * p + i
                pltpu.async_copy(y_hbm.at[iio.at[sl, i, 0]], rows.at[b],
                                 gsems[b])

        def issue_idx(next_blk, next_sl):
            pltpu.async_copy(idx_hbm.at[pl.ds(base + 2 * next_blk, 2)],
                             iio.at[next_sl], isem)

        def issue_scatters(sl, p):
            for i in range(2):
                b = 2 * p + i
                pltpu.async_copy(rows.at[b], accum.at[iio.at[sl, i, 1]],
                                 ssems[b], add=True)

        pltpu.sync_copy(idx_hbm.at[pl.ds(base, 2)], iio.at[0])
        issue_gathers(0, 0)
        issue_idx(1, 1)
        drain_idx()
        issue_gathers(1, 1)
        issue_idx(2, 2)
        drain(g0)
        drain(g1)
        issue_scatters(0, 0)

        def superblock(sb, carry):
            for p in range(2):
                blk = 2 * sb + p
                sl = lax.rem(blk, 3)
                slp = lax.rem(blk - 1, 3)
                q = 1 - p
                drain(ssems[2 * p])
                drain(ssems[2 * p + 1])
                drain_idx()
                issue_gathers(sl, p)
                issue_idx(blk + 1, lax.rem(blk + 1, 3))
                drain(gsems[2 * q])
                drain(gsems[2 * q + 1])
                issue_scatters(slp, q)
            return carry

        lax.fori_loop(1, blocks // 2, superblock, 0)

        drain(g2)
        drain(g3)
        issue_scatters((blocks - 1) % 3, 1)
        for b in range(4):
            drain(ssems[b])
        drain_idx()

        plsc.subcore_barrier()
        pltpu.sync_copy(accum.at[pl.ds(sid * TILE_ROWS, TILE_ROWS)],
                        out_hbm.at[cid, pl.ds(sid * TILE_ROWS, TILE_ROWS)])

    return _sc_gather_scatter


_SC_A = _make_sc(STEPS_A)
_SC_B = _make_sc(STEPS_B)


def _slice_idx(p_in, p_out, k0, nk, steps):
    cap = NW * steps * CH
    kp = nk * P
    npad_pairs = cap - kp
    pad_ramp = jnp.arange(npad_pairs, dtype=jnp.int32)
    iin = (p_in[k0:k0 + nk]
           + (jnp.arange(nk, dtype=jnp.int32) * NPAD)[:, None]).reshape(-1)
    iin = jnp.concatenate([iin, (pad_ramp * 4099) % (nk * NPAD)])
    iout = jnp.concatenate([p_out[k0:k0 + nk].reshape(-1),
                            DUMMY_ROW + pad_ramp % (NPAD - N)])
    idx = jnp.stack([iin.reshape(NW * steps, CH),
                     iout.reshape(NW * steps, CH)], axis=1)
    return jnp.pad(idx, ((0, 2), (0, 0), (0, 0)))


def kernel(feats, pairs_in, pairs_out, W1, g1, b1, m1, v1,
           W2, g2, b2, m2, v2):
    eps = 1e-5
    s1 = g1 * lax.rsqrt(v1 + eps)
    s2 = g2 * lax.rsqrt(v2 + eps)
    w1c = (W1 * s1[None, None, :]).transpose(1, 0, 2).reshape(C, K * C)
    w2c = (W2 * s2[None, None, :]).transpose(1, 0, 2).reshape(C, K * C)
    b1e = (b1 - m1 * s1).reshape(1, C)
    b2e = (b2 - m2 * s2).reshape(1, C)

    featsp = jnp.pad(feats, ((0, NPAD - N), (0, 0)))
    idx_a = _slice_idx(pairs_in, pairs_out, 0, KA, STEPS_A)
    idx_b = _slice_idx(pairs_in, pairs_out, KA, KB, STEPS_B)
    zeros = jnp.zeros((NPAD, C), dtype=jnp.float32)

    y1a = _mm1(featsp, w1c[:, :KA * C], KA).reshape(KA * NPAD, C)
    p1a = _SC_A(y1a, idx_a, zeros)
    y1b = _mm1(featsp, w1c[:, KA * C:], KB).reshape(KB * NPAD, C)
    p1b = _SC_B(y1b, idx_b, zeros)

    y2a = _mm2(p1a, p1b, b1e, w2c[:, :KA * C], KA).reshape(KA * NPAD, C)
    p2a = _SC_A(y2a, idx_a, zeros)
    y2b = _mm2(p1a, p1b, b1e, w2c[:, KA * C:], KB).reshape(KB * NPAD, C)
    p2b = _SC_B(y2b, idx_b, zeros)

    return _final(p2a, p2b, b2e, featsp)


# speedup vs baseline: 1.0239x; 1.0155x over previous
import functools

import jax
import jax.numpy as jnp
from jax import lax
from jax.experimental import pallas as pl
from jax.experimental.pallas import tpu as pltpu
from jax.experimental.pallas import tpu_sc as plsc

N = 10000
C = 128
K = 27
P = 12000

NPAD = 10240
NW = 32
TILE_ROWS = NPAD // 16
DUMMY_ROW = N
CH = 128

KA = 13
KB = K - KA
STEPS_A = 40
STEPS_B = 42

_MM_BM = 1024


def _mm1_body(nk, x_ref, w_ref, o_ref):
    for t in range(nk):
        o_ref[t] = jnp.dot(x_ref[...], w_ref[:, t * C:(t + 1) * C],
                           preferred_element_type=jnp.float32)


def _mm1(x, w, nk):
    return pl.pallas_call(
        functools.partial(_mm1_body, nk),
        grid=(NPAD // _MM_BM,),
        in_specs=[
            pl.BlockSpec((_MM_BM, C), lambda i: (i, 0)),
            pl.BlockSpec((C, nk * C), lambda i: (0, 0)),
        ],
        out_specs=pl.BlockSpec((nk, _MM_BM, C), lambda i: (0, i, 0)),
        out_shape=jax.ShapeDtypeStruct((nk, NPAD, C), jnp.float32),
    )(x, w)


def _mm2_body(nk, pa_ref, pb_ref, b_ref, w_ref, o_ref):
    h = jnp.maximum(pa_ref[0] + pa_ref[1] + pb_ref[0] + pb_ref[1]
                    + b_ref[...], 0.0)
    for t in range(nk):
        o_ref[t] = jnp.dot(h, w_ref[:, t * C:(t + 1) * C],
                           preferred_element_type=jnp.float32)


def _mm2(pa, pb, b, w, nk):
    pblk = pl.BlockSpec((2, _MM_BM, C), lambda i: (0, i, 0))
    return pl.pallas_call(
        functools.partial(_mm2_body, nk),
        grid=(NPAD // _MM_BM,),
        in_specs=[
            pblk, pblk,
            pl.BlockSpec((1, C), lambda i: (0, 0)),
            pl.BlockSpec((C, nk * C), lambda i: (0, 0)),
        ],
        out_specs=pl.BlockSpec((nk, _MM_BM, C), lambda i: (0, i, 0)),
        out_shape=jax.ShapeDtypeStruct((nk, NPAD, C), jnp.float32),
    )(pa, pb, b, w)


def _final_body(pa_ref, pb_ref, b_ref, f_ref, o_ref):
    o_ref[...] = jnp.maximum(
        pa_ref[0] + pa_ref[1] + pb_ref[0] + pb_ref[1]
        + b_ref[...] + f_ref[...], 0.0)


def _final(pa, pb, b, f):
    blk = pl.BlockSpec((1000, C), lambda i: (i, 0))
    pblk = pl.BlockSpec((2, 1000, C), lambda i: (0, i, 0))
    return pl.pallas_call(
        _final_body,
        grid=(N // 1000,),
        in_specs=[pblk, pblk,
                  pl.BlockSpec((1, C), lambda i: (0, 0)), blk],
        out_specs=blk,
        out_shape=jax.ShapeDtypeStruct((N, C), jnp.float32),
    )(pa, pb, b, f)


def _make_sc(steps):

    @functools.partial(
        pl.kernel,
        out_type=jax.ShapeDtypeStruct((2, NPAD, C), jnp.float32),
        mesh=plsc.VectorSubcoreMesh(core_axis_name="c",
                                    subcore_axis_name="s"),
        scratch_types=[
            pltpu.VMEM((3, 2, CH), jnp.int32),
            pltpu.VMEM((2, CH, C), jnp.float32),
            pltpu.VMEM_SHARED((NPAD, C), jnp.float32),
            pltpu.SemaphoreType.DMA,
            pltpu.SemaphoreType.DMA,
            pltpu.SemaphoreType.DMA,
            pltpu.SemaphoreType.DMA,
            pltpu.SemaphoreType.DMA,
        ],
    )
    def _sc_gather_scatter(y_hbm, idx_hbm, zeros_hbm, out_hbm,
                           iio, rows, accum, isem, g0, g1, s0, s1):
        gsems = (g0, g1)
        ssems = (s0, s1)
        cid = lax.axis_index("c")
        sid = lax.axis_index("s")
        wid = sid * 2 + cid

        pltpu.sync_copy(zeros_hbm.at[pl.ds(sid * TILE_ROWS, TILE_ROWS)],
                        accum.at[pl.ds(sid * TILE_ROWS, TILE_ROWS)])
        plsc.subcore_barrier()

        base = wid * steps

        def drain(sem_b):
            pltpu.make_async_copy(y_hbm.at[pl.ds(0, CH)], rows.at[0],
                                  sem_b).wait()

        def drain_idx():
            pltpu.make_async_copy(idx_hbm.at[pl.ds(0, 1)],
                                  iio.at[pl.ds(0, 1)], isem).wait()

        def issue_gather(sl, p):
            pltpu.async_copy(y_hbm.at[iio.at[sl, 0]], rows.at[p], gsems[p])

        def issue_idx(next_blk, next_sl):
            pltpu.async_copy(idx_hbm.at[base + next_blk], iio.at[next_sl],
                             isem)

        def issue_scatter(sl, p):
            pltpu.async_copy(rows.at[p], accum.at[iio.at[sl, 1]],
                             ssems[p], add=True)

        pltpu.sync_copy(idx_hbm.at[base], iio.at[0])
        issue_gather(0, 0)
        issue_idx(1, 1)
        drain_idx()
        issue_gather(1, 1)
        issue_idx(2, 2)
        drain(g0)
        issue_scatter(0, 0)

        def superblock(sb, carry):
            for p in range(2):
                blk = 2 * sb + p
                sl = lax.rem(blk, 3)
                slp = lax.rem(blk - 1, 3)
                q = 1 - p
                drain(ssems[p])
                drain_idx()
                issue_gather(sl, p)
                issue_idx(blk + 1, lax.rem(blk + 1, 3))
                drain(gsems[q])
                issue_scatter(slp, q)
            return carry

        lax.fori_loop(1, steps // 2, superblock, 0)

        drain(g1)
        issue_scatter((steps - 1) % 3, 1)
        drain(s0)
        drain(s1)
        drain_idx()

        plsc.subcore_barrier()
        pltpu.sync_copy(accum.at[pl.ds(sid * TILE_ROWS, TILE_ROWS)],
                        out_hbm.at[cid, pl.ds(sid * TILE_ROWS, TILE_ROWS)])

    return _sc_gather_scatter


_SC_A = _make_sc(STEPS_A)
_SC_B = _make_sc(STEPS_B)


def _slice_idx(p_in, p_out, k0, nk, steps):
    cap = NW * steps * CH
    kp = nk * P
    npad_pairs = cap - kp
    pad_ramp = jnp.arange(npad_pairs, dtype=jnp.int32)
    iin = (p_in[k0:k0 + nk]
           + (jnp.arange(nk, dtype=jnp.int32) * NPAD)[:, None]).reshape(-1)
    iin = jnp.concatenate([iin, (pad_ramp * 4099) % (nk * NPAD)])
    iout = jnp.concatenate([p_out[k0:k0 + nk].reshape(-1),
                            DUMMY_ROW + pad_ramp % (NPAD - N)])
    idx = jnp.stack([iin.reshape(NW * steps, CH),
                     iout.reshape(NW * steps, CH)], axis=1)
    return jnp.pad(idx, ((0, 2), (0, 0), (0, 0)))


def kernel(feats, pairs_in, pairs_out, W1, g1, b1, m1, v1,
           W2, g2, b2, m2, v2):
    eps = 1e-5
    s1 = g1 * lax.rsqrt(v1 + eps)
    s2 = g2 * lax.rsqrt(v2 + eps)
    w1c = (W1 * s1[None, None, :]).transpose(1, 0, 2).reshape(C, K * C)
    w2c = (W2 * s2[None, None, :]).transpose(1, 0, 2).reshape(C, K * C)
    b1e = (b1 - m1 * s1).reshape(1, C)
    b2e = (b2 - m2 * s2).reshape(1, C)

    featsp = jnp.pad(feats, ((0, NPAD - N), (0, 0)))
    idx_a = _slice_idx(pairs_in, pairs_out, 0, KA, STEPS_A)
    idx_b = _slice_idx(pairs_in, pairs_out, KA, KB, STEPS_B)
    zeros = jnp.zeros((NPAD, C), dtype=jnp.float32)

    y1a = _mm1(featsp, w1c[:, :KA * C], KA).reshape(KA * NPAD, C)
    p1a = _SC_A(y1a, idx_a, zeros)
    y1b = _mm1(featsp, w1c[:, KA * C:], KB).reshape(KB * NPAD, C)
    p1b = _SC_B(y1b, idx_b, zeros)

    y2a = _mm2(p1a, p1b, b1e, w2c[:, :KA * C], KA).reshape(KA * NPAD, C)
    p2a = _SC_A(y2a, idx_a, zeros)
    y2b = _mm2(p1a, p1b, b1e, w2c[:, KA * C:], KB).reshape(KB * NPAD, C)
    p2b = _SC_B(y2b, idx_b, zeros)

    return _final(p2a, p2b, b2e, featsp)


# speedup vs baseline: 1.0460x; 1.0216x over previous
import functools

import jax
import jax.numpy as jnp
from jax import lax
from jax.experimental import pallas as pl
from jax.experimental.pallas import tpu as pltpu
from jax.experimental.pallas import tpu_sc as plsc

N = 10000
C = 128
K = 27
P = 12000

NPAD = 10240
NW = 32
TILE_ROWS = NPAD // 16
DUMMY_ROW = N
CH = 128

KA = 12
KB = K - KA
STEPS_A = 36
STEPS_B = 44

_MM_BM = 1024


def _mm1_body(nk, x_ref, w_ref, o_ref):
    for t in range(nk):
        o_ref[t] = jnp.dot(x_ref[...], w_ref[:, t * C:(t + 1) * C],
                           preferred_element_type=jnp.float32)


def _mm1(x, w, nk):
    return pl.pallas_call(
        functools.partial(_mm1_body, nk),
        grid=(NPAD // _MM_BM,),
        in_specs=[
            pl.BlockSpec((_MM_BM, C), lambda i: (i, 0)),
            pl.BlockSpec((C, nk * C), lambda i: (0, 0)),
        ],
        out_specs=pl.BlockSpec((nk, _MM_BM, C), lambda i: (0, i, 0)),
        out_shape=jax.ShapeDtypeStruct((nk, NPAD, C), jnp.float32),
    )(x, w)


def _mm2_body(nk, pa_ref, pb_ref, b_ref, w_ref, o_ref):
    h = jnp.maximum(pa_ref[0] + pa_ref[1] + pb_ref[0] + pb_ref[1]
                    + b_ref[...], 0.0)
    for t in range(nk):
        o_ref[t] = jnp.dot(h, w_ref[:, t * C:(t + 1) * C],
                           preferred_element_type=jnp.float32)


def _mm2(pa, pb, b, w, nk):
    pblk = pl.BlockSpec((2, _MM_BM, C), lambda i: (0, i, 0))
    return pl.pallas_call(
        functools.partial(_mm2_body, nk),
        grid=(NPAD // _MM_BM,),
        in_specs=[
            pblk, pblk,
            pl.BlockSpec((1, C), lambda i: (0, 0)),
            pl.BlockSpec((C, nk * C), lambda i: (0, 0)),
        ],
        out_specs=pl.BlockSpec((nk, _MM_BM, C), lambda i: (0, i, 0)),
        out_shape=jax.ShapeDtypeStruct((nk, NPAD, C), jnp.float32),
    )(pa, pb, b, w)


def _final_body(pa_ref, pb_ref, b_ref, f_ref, o_ref):
    o_ref[...] = jnp.maximum(
        pa_ref[0] + pa_ref[1] + pb_ref[0] + pb_ref[1]
        + b_ref[...] + f_ref[...], 0.0)


def _final(pa, pb, b, f):
    blk = pl.BlockSpec((1000, C), lambda i: (i, 0))
    pblk = pl.BlockSpec((2, 1000, C), lambda i: (0, i, 0))
    return pl.pallas_call(
        _final_body,
        grid=(N // 1000,),
        in_specs=[pblk, pblk,
                  pl.BlockSpec((1, C), lambda i: (0, 0)), blk],
        out_specs=blk,
        out_shape=jax.ShapeDtypeStruct((N, C), jnp.float32),
    )(pa, pb, b, f)


def _make_sc(steps):

    @functools.partial(
        pl.kernel,
        out_type=jax.ShapeDtypeStruct((2, NPAD, C), jnp.float32),
        mesh=plsc.VectorSubcoreMesh(core_axis_name="c",
                                    subcore_axis_name="s"),
        scratch_types=[
            pltpu.VMEM((3, 2, CH), jnp.int32),
            pltpu.VMEM((2, CH, C), jnp.float32),
            pltpu.VMEM_SHARED((NPAD, C), jnp.float32),
            pltpu.SemaphoreType.DMA,
            pltpu.SemaphoreType.DMA,
            pltpu.SemaphoreType.DMA,
            pltpu.SemaphoreType.DMA,
            pltpu.SemaphoreType.DMA,
        ],
    )
    def _sc_gather_scatter(y_hbm, idx_hbm, zeros_hbm, out_hbm,
                           iio, rows, accum, isem, g0, g1, s0, s1):
        gsems = (g0, g1)
        ssems = (s0, s1)
        cid = lax.axis_index("c")
        sid = lax.axis_index("s")
        wid = sid * 2 + cid

        pltpu.sync_copy(zeros_hbm.at[pl.ds(sid * TILE_ROWS, TILE_ROWS)],
                        accum.at[pl.ds(sid * TILE_ROWS, TILE_ROWS)])
        plsc.subcore_barrier()

        base = wid * steps

        def drain(sem_b):
            pltpu.make_async_copy(y_hbm.at[pl.ds(0, CH)], rows.at[0],
                                  sem_b).wait()

        def drain_idx():
            pltpu.make_async_copy(idx_hbm.at[pl.ds(0, 1)],
                                  iio.at[pl.ds(0, 1)], isem).wait()

        def issue_gather(sl, p):
            pltpu.async_copy(y_hbm.at[iio.at[sl, 0]], rows.at[p], gsems[p])

        def issue_idx(next_blk, next_sl):
            pltpu.async_copy(idx_hbm.at[base + next_blk], iio.at[next_sl],
                             isem)

        def issue_scatter(sl, p):
            pltpu.async_copy(rows.at[p], accum.at[iio.at[sl, 1]],
                             ssems[p], add=True)

        pltpu.sync_copy(idx_hbm.at[base], iio.at[0])
        issue_gather(0, 0)
        issue_idx(1, 1)
        drain_idx()
        issue_gather(1, 1)
        issue_idx(2, 2)
        drain(g0)
        issue_scatter(0, 0)

        def superblock(sb, carry):
            for p in range(2):
                blk = 2 * sb + p
                sl = lax.rem(blk, 3)
                slp = lax.rem(blk - 1, 3)
                q = 1 - p
                drain(ssems[p])
                drain_idx()
                issue_gather(sl, p)
                issue_idx(blk + 1, lax.rem(blk + 1, 3))
                drain(gsems[q])
                issue_scatter(slp, q)
            return carry

        lax.fori_loop(1, steps // 2, superblock, 0)

        drain(g1)
        issue_scatter((steps - 1) % 3, 1)
        drain(s0)
        drain(s1)
        drain_idx()

        plsc.subcore_barrier()
        pltpu.sync_copy(accum.at[pl.ds(sid * TILE_ROWS, TILE_ROWS)],
                        out_hbm.at[cid, pl.ds(sid * TILE_ROWS, TILE_ROWS)])

    return _sc_gather_scatter


_SC_A = _make_sc(STEPS_A)
_SC_B = _make_sc(STEPS_B)


def _slice_idx(p_in, p_out, k0, nk, steps):
    cap = NW * steps * CH
    kp = nk * P
    npad_pairs = cap - kp
    pad_ramp = jnp.arange(npad_pairs, dtype=jnp.int32)
    iin = (p_in[k0:k0 + nk]
           + (jnp.arange(nk, dtype=jnp.int32) * NPAD)[:, None]).reshape(-1)
    iin = jnp.concatenate([iin, (pad_ramp * 4099) % (nk * NPAD)])
    iout = jnp.concatenate([p_out[k0:k0 + nk].reshape(-1),
                            DUMMY_ROW + pad_ramp % (NPAD - N)])
    idx = jnp.stack([iin.reshape(NW * steps, CH),
                     iout.reshape(NW * steps, CH)], axis=1)
    return jnp.pad(idx, ((0, 2), (0, 0), (0, 0)))


def kernel(feats, pairs_in, pairs_out, W1, g1, b1, m1, v1,
           W2, g2, b2, m2, v2):
    eps = 1e-5
    s1 = g1 * lax.rsqrt(v1 + eps)
    s2 = g2 * lax.rsqrt(v2 + eps)
    w1c = (W1 * s1[None, None, :]).transpose(1, 0, 2).reshape(C, K * C)
    w2c = (W2 * s2[None, None, :]).transpose(1, 0, 2).reshape(C, K * C)
    b1e = (b1 - m1 * s1).reshape(1, C)
    b2e = (b2 - m2 * s2).reshape(1, C)

    featsp = jnp.pad(feats, ((0, NPAD - N), (0, 0)))
    idx_a = _slice_idx(pairs_in, pairs_out, 0, KA, STEPS_A)
    idx_b = _slice_idx(pairs_in, pairs_out, KA, KB, STEPS_B)
    zeros = jnp.zeros((NPAD, C), dtype=jnp.float32)

    y1a = _mm1(featsp, w1c[:, :KA * C], KA).reshape(KA * NPAD, C)
    p1a = _SC_A(y1a, idx_a, zeros)
    y1b = _mm1(featsp, w1c[:, KA * C:], KB).reshape(KB * NPAD, C)
    p1b = _SC_B(y1b, idx_b, zeros)

    y2a = _mm2(p1a, p1b, b1e, w2c[:, :KA * C], KA).reshape(KA * NPAD, C)
    p2a = _SC_A(y2a, idx_a, zeros)
    y2b = _mm2(p1a, p1b, b1e, w2c[:, KA * C:], KB).reshape(KB * NPAD, C)
    p2b = _SC_B(y2b, idx_b, zeros)

    return _final(p2a, p2b, b2e, featsp)


# speedup vs baseline: 1.0560x; 1.0096x over previous
import functools

import jax
import jax.numpy as jnp
from jax import lax
from jax.experimental import pallas as pl
from jax.experimental.pallas import tpu as pltpu
from jax.experimental.pallas import tpu_sc as plsc

N = 10000
C = 128
K = 27
P = 12000

NPAD = 10240
NW = 32
TILE_ROWS = NPAD // 16
DUMMY_ROW = N
CH = 128

KA = 12
KB = K - KA
STEPS_A = 36
STEPS_B = 44

_MM_BM = 1024


def _mm1_body(nk, x_ref, w_ref, o_ref):
    for t in range(nk):
        o_ref[t] = jnp.dot(x_ref[...], w_ref[:, t * C:(t + 1) * C],
                           preferred_element_type=jnp.float32)


def _mm1(x, w, nk):
    return pl.pallas_call(
        functools.partial(_mm1_body, nk),
        grid=(NPAD // _MM_BM,),
        in_specs=[
            pl.BlockSpec((_MM_BM, C), lambda i: (i, 0)),
            pl.BlockSpec((C, nk * C), lambda i: (0, 0)),
        ],
        out_specs=pl.BlockSpec((nk, _MM_BM, C), lambda i: (0, i, 0)),
        out_shape=jax.ShapeDtypeStruct((nk, NPAD, C), jnp.float32),
    )(x, w)


def _mm2_body(nk, pa_ref, pb_ref, b_ref, w_ref, o_ref):
    h = jnp.maximum(pa_ref[0] + pa_ref[1] + pb_ref[0] + pb_ref[1]
                    + b_ref[...], 0.0)
    for t in range(nk):
        o_ref[t] = jnp.dot(h, w_ref[:, t * C:(t + 1) * C],
                           preferred_element_type=jnp.float32)


def _mm2(pa, pb, b, w, nk):
    pblk = pl.BlockSpec((2, _MM_BM, C), lambda i: (0, i, 0))
    return pl.pallas_call(
        functools.partial(_mm2_body, nk),
        grid=(NPAD // _MM_BM,),
        in_specs=[
            pblk, pblk,
            pl.BlockSpec((1, C), lambda i: (0, 0)),
            pl.BlockSpec((C, nk * C), lambda i: (0, 0)),
        ],
        out_specs=pl.BlockSpec((nk, _MM_BM, C), lambda i: (0, i, 0)),
        out_shape=jax.ShapeDtypeStruct((nk, NPAD, C), jnp.float32),
    )(pa, pb, b, w)


def _final_body(pa_ref, pb_ref, b_ref, f_ref, o_ref):
    o_ref[...] = jnp.maximum(
        pa_ref[0] + pa_ref[1] + pb_ref[0] + pb_ref[1]
        + b_ref[...] + f_ref[...], 0.0)


def _final(pa, pb, b, f):
    blk = pl.BlockSpec((1000, C), lambda i: (i, 0))
    pblk = pl.BlockSpec((2, 1000, C), lambda i: (0, i, 0))
    return pl.pallas_call(
        _final_body,
        grid=(N // 1000,),
        in_specs=[pblk, pblk,
                  pl.BlockSpec((1, C), lambda i: (0, 0)), blk],
        out_specs=blk,
        out_shape=jax.ShapeDtypeStruct((N, C), jnp.float32),
    )(pa, pb, b, f)


def _make_sc(steps):

    @functools.partial(
        pl.kernel,
        out_type=jax.ShapeDtypeStruct((2, NPAD, C), jnp.float32),
        mesh=plsc.VectorSubcoreMesh(core_axis_name="c",
                                    subcore_axis_name="s"),
        scratch_types=[
            pltpu.VMEM((3, 2, CH), jnp.int32),
            pltpu.VMEM((2, CH, C), jnp.float32),
            pltpu.VMEM_SHARED((NPAD, C), jnp.float32),
            pltpu.SemaphoreType.DMA,
            pltpu.SemaphoreType.DMA,
            pltpu.SemaphoreType.DMA,
            pltpu.SemaphoreType.DMA,
            pltpu.SemaphoreType.DMA,
        ],
    )
    def _sc_gather_scatter(y_hbm, idx_hbm, zeros_hbm, out_hbm,
                           iio, rows, accum, isem, g0, g1, s0, s1):
        gsems = (g0, g1)
        ssems = (s0, s1)
        cid = lax.axis_index("c")
        sid = lax.axis_index("s")
        wid = sid * 2 + cid

        base = wid * steps

        def drain(sem_b):
            pltpu.make_async_copy(y_hbm.at[pl.ds(0, CH)], rows.at[0],
                                  sem_b).wait()

        def drain_idx():
            pltpu.make_async_copy(idx_hbm.at[pl.ds(0, 1)],
                                  iio.at[pl.ds(0, 1)], isem).wait()

        def issue_gather(sl, p):
            pltpu.async_copy(y_hbm.at[iio.at[sl, 0]], rows.at[p], gsems[p])

        def issue_idx(next_blk, next_sl):
            pltpu.async_copy(idx_hbm.at[base + next_blk], iio.at[next_sl],
                             isem)

        def issue_scatter(sl, p):
            pltpu.async_copy(rows.at[p], accum.at[iio.at[sl, 1]],
                             ssems[p], add=True)

        pltpu.sync_copy(idx_hbm.at[base], iio.at[0])
        issue_gather(0, 0)
        issue_idx(1, 1)
        pltpu.sync_copy(zeros_hbm.at[pl.ds(sid * TILE_ROWS, TILE_ROWS)],
                        accum.at[pl.ds(sid * TILE_ROWS, TILE_ROWS)])
        plsc.subcore_barrier()
        drain_idx()
        issue_gather(1, 1)
        issue_idx(2, 2)
        drain(g0)
        issue_scatter(0, 0)

        def superblock(sb, carry):
            for p in range(2):
                blk = 2 * sb + p
                sl = lax.rem(blk, 3)
                slp = lax.rem(blk - 1, 3)
                q = 1 - p
                drain(ssems[p])
                drain_idx()
                issue_gather(sl, p)
                issue_idx(blk + 1, lax.rem(blk + 1, 3))
                drain(gsems[q])
                issue_scatter(slp, q)
            return carry

        lax.fori_loop(1, steps // 2, superblock, 0)

        drain(g1)
        issue_scatter((steps - 1) % 3, 1)
        drain(s0)
        drain(s1)
        drain_idx()

        plsc.subcore_barrier()
        pltpu.sync_copy(accum.at[pl.ds(sid * TILE_ROWS, TILE_ROWS)],
                        out_hbm.at[cid, pl.ds(sid * TILE_ROWS, TILE_ROWS)])

    return _sc_gather_scatter


_SC_A = _make_sc(STEPS_A)
_SC_B = _make_sc(STEPS_B)


def _slice_idx(p_in, p_out, k0, nk, steps):
    cap = NW * steps * CH
    kp = nk * P
    npad_pairs = cap - kp
    pad_ramp = jnp.arange(npad_pairs, dtype=jnp.int32)
    iin = (p_in[k0:k0 + nk]
           + (jnp.arange(nk, dtype=jnp.int32) * NPAD)[:, None]).reshape(-1)
    iin = jnp.concatenate([iin, (pad_ramp * 4099) % (nk * NPAD)])
    iout = jnp.concatenate([p_out[k0:k0 + nk].reshape(-1),
                            DUMMY_ROW + pad_ramp % (NPAD - N)])
    idx = jnp.stack([iin.reshape(NW * steps, CH),
                     iout.reshape(NW * steps, CH)], axis=1)
    return jnp.pad(idx, ((0, 2), (0, 0), (0, 0)))


def kernel(feats, pairs_in, pairs_out, W1, g1, b1, m1, v1,
           W2, g2, b2, m2, v2):
    eps = 1e-5
    s1 = g1 * lax.rsqrt(v1 + eps)
    s2 = g2 * lax.rsqrt(v2 + eps)
    w1c = (W1 * s1[None, None, :]).transpose(1, 0, 2).reshape(C, K * C)
    w2c = (W2 * s2[None, None, :]).transpose(1, 0, 2).reshape(C, K * C)
    b1e = (b1 - m1 * s1).reshape(1, C)
    b2e = (b2 - m2 * s2).reshape(1, C)

    featsp = jnp.pad(feats, ((0, NPAD - N), (0, 0)))
    idx_a = _slice_idx(pairs_in, pairs_out, 0, KA, STEPS_A)
    idx_b = _slice_idx(pairs_in, pairs_out, KA, KB, STEPS_B)
    zeros = jnp.zeros((NPAD, C), dtype=jnp.float32)

    y1a = _mm1(featsp, w1c[:, :KA * C], KA).reshape(KA * NPAD, C)
    p1a = _SC_A(y1a, idx_a, zeros)
    y1b = _mm1(featsp, w1c[:, KA * C:], KB).reshape(KB * NPAD, C)
    p1b = _SC_B(y1b, idx_b, zeros)

    y2a = _mm2(p1a, p1b, b1e, w2c[:, :KA * C], KA).reshape(KA * NPAD, C)
    p2a = _SC_A(y2a, idx_a, zeros)
    y2b = _mm2(p1a, p1b, b1e, w2c[:, KA * C:], KB).reshape(KB * NPAD, C)
    p2b = _SC_B(y2b, idx_b, zeros)

    return _final(p2a, p2b, b2e, featsp)


# speedup vs baseline: 1.0691x; 1.0124x over previous
import functools

import jax
import jax.numpy as jnp
from jax import lax
from jax.experimental import pallas as pl
from jax.experimental.pallas import tpu as pltpu
from jax.experimental.pallas import tpu_sc as plsc

N = 10000
C = 128
K = 27
P = 12000

NPAD = 10240
NW = 32
TILE_ROWS = NPAD // 16
DUMMY_ROW = N
CH = 128

KA = 12
KB = K - KA
STEPS_A = 36
STEPS_B = 44

_MM_BM = 1024


def _mm1_body(nk, x_ref, w_ref, o_ref):
    for t in range(nk):
        o_ref[t] = jnp.dot(x_ref[...], w_ref[:, t * C:(t + 1) * C],
                           preferred_element_type=jnp.float32)


def _mm1(x, w, nk):
    return pl.pallas_call(
        functools.partial(_mm1_body, nk),
        grid=(NPAD // _MM_BM,),
        in_specs=[
            pl.BlockSpec((_MM_BM, C), lambda i: (i, 0)),
            pl.BlockSpec((C, nk * C), lambda i: (0, 0)),
        ],
        out_specs=pl.BlockSpec((nk, _MM_BM, C), lambda i: (0, i, 0)),
        out_shape=jax.ShapeDtypeStruct((nk, NPAD, C), jnp.float32),
    )(x, w)


def _mm2_body(nk, pa_ref, pb_ref, b_ref, w_ref, o_ref):
    h = jnp.maximum(pa_ref[0] + pa_ref[1] + pb_ref[0] + pb_ref[1]
                    + b_ref[...], 0.0)
    for t in range(nk):
        o_ref[t] = jnp.dot(h, w_ref[:, t * C:(t + 1) * C],
                           preferred_element_type=jnp.float32)


def _mm2(pa, pb, b, w, nk):
    pblk = pl.BlockSpec((2, _MM_BM, C), lambda i: (0, i, 0))
    return pl.pallas_call(
        functools.partial(_mm2_body, nk),
        grid=(NPAD // _MM_BM,),
        in_specs=[
            pblk, pblk,
            pl.BlockSpec((1, C), lambda i: (0, 0)),
            pl.BlockSpec((C, nk * C), lambda i: (0, 0)),
        ],
        out_specs=pl.BlockSpec((nk, _MM_BM, C), lambda i: (0, i, 0)),
        out_shape=jax.ShapeDtypeStruct((nk, NPAD, C), jnp.float32),
    )(pa, pb, b, w)


def _final_body(pa_ref, pb_ref, b_ref, f_ref, o_ref):
    o_ref[...] = jnp.maximum(
        pa_ref[0] + pa_ref[1] + pb_ref[0] + pb_ref[1]
        + b_ref[...] + f_ref[...], 0.0)


def _final(pa, pb, b, f):
    blk = pl.BlockSpec((1000, C), lambda i: (i, 0))
    pblk = pl.BlockSpec((2, 1000, C), lambda i: (0, i, 0))
    return pl.pallas_call(
        _final_body,
        grid=(N // 1000,),
        in_specs=[pblk, pblk,
                  pl.BlockSpec((1, C), lambda i: (0, 0)), blk],
        out_specs=blk,
        out_shape=jax.ShapeDtypeStruct((N, C), jnp.float32),
    )(pa, pb, b, f)


def _make_sc(steps):

    @functools.partial(
        pl.kernel,
        out_type=jax.ShapeDtypeStruct((2, NPAD, C), jnp.float32),
        mesh=plsc.VectorSubcoreMesh(core_axis_name="c",
                                    subcore_axis_name="s"),
        scratch_types=[
            pltpu.VMEM((3, 2, CH), jnp.int32),
            pltpu.VMEM((2, CH, C), jnp.float32),
            pltpu.VMEM_SHARED((NPAD, C), jnp.float32),
            pltpu.SemaphoreType.DMA,
            pltpu.SemaphoreType.DMA,
            pltpu.SemaphoreType.DMA,
            pltpu.SemaphoreType.DMA,
            pltpu.SemaphoreType.DMA,
        ],
    )
    def _sc_gather_scatter(y_hbm, idx_hbm, zeros_hbm, out_hbm,
                           iio, rows, accum, isem, g0, g1, s0, s1):
        gsems = (g0, g1)
        ssems = (s0, s1)
        cid = lax.axis_index("c")
        sid = lax.axis_index("s")
        wid = sid * 2 + cid

        base = wid * steps

        def drain(sem_b):
            pltpu.make_async_copy(y_hbm.at[pl.ds(0, CH)], rows.at[0],
                                  sem_b).wait()

        def drain_idx():
            pltpu.make_async_copy(idx_hbm.at[pl.ds(0, 1)],
                                  iio.at[pl.ds(0, 1)], isem).wait()

        def issue_gather(sl, p):
            pltpu.async_copy(y_hbm.at[iio.at[sl, 0]], rows.at[p], gsems[p])

        def issue_idx(next_blk, next_sl):
            pltpu.async_copy(idx_hbm.at[base + next_blk], iio.at[next_sl],
                             isem)

        def issue_scatter(sl, p):
            pltpu.async_copy(rows.at[p], accum.at[iio.at[sl, 1]],
                             ssems[p], add=True)

        pltpu.sync_copy(idx_hbm.at[base], iio.at[0])
        issue_gather(0, 0)
        issue_idx(1, 1)
        pltpu.sync_copy(zeros_hbm.at[pl.ds(sid * TILE_ROWS, TILE_ROWS)],
                        accum.at[pl.ds(sid * TILE_ROWS, TILE_ROWS)])
        plsc.subcore_barrier()
        drain_idx()
        issue_gather(1, 1)
        issue_idx(2, 2)
        drain(g0)
        issue_scatter(0, 0)

        def superblock(sb, carry):
            for p in range(2):
                blk = 2 * sb + p
                sl = lax.rem(blk, 3)
                slp = lax.rem(blk - 1, 3)
                q = 1 - p
                drain(ssems[p])
                drain_idx()
                issue_gather(sl, p)
                issue_idx(blk + 1, lax.rem(blk + 1, 3))
                drain(gsems[q])
                issue_scatter(slp, q)
            return carry

        lax.fori_loop(1, steps // 2, superblock, 0)

        drain(g1)
        issue_scatter((steps - 1) % 3, 1)
        drain(s0)
        drain(s1)
        drain_idx()

        plsc.subcore_barrier()
        pltpu.sync_copy(accum.at[pl.ds(sid * TILE_ROWS, TILE_ROWS)],
                        out_hbm.at[cid, pl.ds(sid * TILE_ROWS, TILE_ROWS)])

    return _sc_gather_scatter


_SC_A = _make_sc(STEPS_A)
_SC_B = _make_sc(STEPS_B)


def _slice_idx(p_in, p_out, k0, nk, steps):
    cap = NW * steps * CH
    kp = nk * P
    npad_pairs = cap - kp
    pad_ramp = jnp.arange(npad_pairs, dtype=jnp.int32)
    iin = (p_in[k0:k0 + nk]
           + (jnp.arange(nk, dtype=jnp.int32) * NPAD)[:, None]).reshape(-1)
    iin = jnp.concatenate([iin, (pad_ramp * 4099) % (nk * NPAD)])
    iout = jnp.concatenate([p_out[k0:k0 + nk].reshape(-1),
                            DUMMY_ROW + pad_ramp % (NPAD - N)])
    idx = jnp.stack([iin.reshape(NW * steps, CH),
                     iout.reshape(NW * steps, CH)], axis=1)
    return jnp.pad(idx, ((0, 2), (0, 0), (0, 0)))


def kernel(feats, pairs_in, pairs_out, W1, g1, b1, m1, v1,
           W2, g2, b2, m2, v2):
    eps = 1e-5
    s1 = g1 * lax.rsqrt(v1 + eps)
    s2 = g2 * lax.rsqrt(v2 + eps)
    w1c = (W1 * s1[None, None, :]).transpose(1, 0, 2).reshape(C, K * C)
    w2c = (W2 * s2[None, None, :]).transpose(1, 0, 2).reshape(C, K * C)
    b1e = (b1 - m1 * s1).reshape(1, C)
    b2e = (b2 - m2 * s2).reshape(1, C)

    idx_a = _slice_idx(pairs_in, pairs_out, 0, KA, STEPS_A)
    idx_b = _slice_idx(pairs_in, pairs_out, KA, KB, STEPS_B)
    zeros = jnp.zeros((NPAD, C), dtype=jnp.float32)

    y1a = _mm1(feats, w1c[:, :KA * C], KA).reshape(KA * NPAD, C)
    p1a = _SC_A(y1a, idx_a, zeros)
    y1b = _mm1(feats, w1c[:, KA * C:], KB).reshape(KB * NPAD, C)
    p1b = _SC_B(y1b, idx_b, zeros)

    y2a = _mm2(p1a, p1b, b1e, w2c[:, :KA * C], KA).reshape(KA * NPAD, C)
    p2a = _SC_A(y2a, idx_a, zeros)
    y2b = _mm2(p1a, p1b, b1e, w2c[:, KA * C:], KB).reshape(KB * NPAD, C)
    p2b = _SC_B(y2b, idx_b, zeros)

    return _final(p2a, p2b, b2e, feats)
